# baseline (speedup 1.0000x reference)
import functools

import jax
import jax.numpy as jnp
from jax import lax
from jax.experimental import pallas as pl
from jax.experimental.pallas import tpu as pltpu
from jax.experimental.pallas import tpu_sc as plsc

_EMBED = 32
_IDX_CHUNK = 128


def _dot_kernel(uid_hbm, mid_hbm, utab_hbm, mtab_hbm, out_hbm,
                uid_v, mid_v, uidx_v, midx_v, stu_v, stm_v, out_v,
                sem0, sem1, *, b_per_w, num_cores):
    wid = lax.axis_index("s") * num_cores + lax.axis_index("c")
    base = wid * b_per_w
    n_chunks = b_per_w // _IDX_CHUNK
    sems = (sem0, sem1)

    pltpu.sync_copy(uid_hbm.at[pl.ds(base, b_per_w)], uid_v)
    pltpu.sync_copy(mid_hbm.at[pl.ds(base, b_per_w)], mid_v)

    for j in range(n_chunks):
        for i in range(_IDX_CHUNK // 16):
            sl = pl.ds(j * _IDX_CHUNK + i * 16, 16)
            uidx_v[j, pl.ds(i * 16, 16)] = lax.shift_right_logical(
                uid_v[sl], 2)
            midx_v[j, pl.ds(i * 16, 16)] = lax.shift_right_logical(
                mid_v[sl], 2)

    def fire(j):
        s = j & 1
        return (
            pltpu.async_copy(utab_hbm.at[uidx_v.at[j]], stu_v.at[s], sems[s]),
            pltpu.async_copy(mtab_hbm.at[midx_v.at[j]], stm_v.at[s], sems[s]),
        )

    lane = lax.iota(jnp.int32, 16)

    def compute_chunk(j):
        s = j & 1
        stu = stu_v.at[s]
        stm = stm_v.at[s]

        def group(g, _):
            k0 = g * 16
            uv = uid_v[pl.ds(j * _IDX_CHUNK + k0, 16)]
            mv = mid_v[pl.ds(j * _IDX_CHUNK + k0, 16)]
            offu = (uv & 3) * _EMBED
            offm = (mv & 3) * _EMBED
            rows = k0 + lane
            acc = jnp.zeros((16,), jnp.float32)
            cu = lane & (_EMBED - 1)
            for _d in range(_EMBED):
                u = plsc.load_gather(stu, [rows, offu + cu])
                m = plsc.load_gather(stm, [rows, offm + cu])
                acc = acc + u * m
                cu = (cu + 1) & (_EMBED - 1)
            out_v[pl.ds(j * _IDX_CHUNK + k0, 16)] = acc
            return 0

        lax.fori_loop(0, _IDX_CHUNK // 16, group, 0)

    inflight = fire(0)
    for j in range(n_chunks):
        nxt = fire(j + 1) if j + 1 < n_chunks else None
        for c in inflight:
            c.wait()
        compute_chunk(j)
        inflight = nxt

    pltpu.sync_copy(out_v, out_hbm.at[pl.ds(base, b_per_w)])


def kernel(user_ids, movie_ids, user_table, movie_table):
    batch = user_ids.shape[0]
    info = plsc.get_sparse_core_info()
    nw = info.num_cores * info.num_subcores
    b_per_w = batch // nw
    mesh = plsc.VectorSubcoreMesh(core_axis_name="c", subcore_axis_name="s")

    ut = user_table.reshape(-1, 128)
    mt = movie_table.reshape(-1, 128)

    run = pl.kernel(
        functools.partial(_dot_kernel, b_per_w=b_per_w,
                          num_cores=info.num_cores),
        mesh=mesh,
        compiler_params=pltpu.CompilerParams(needs_layout_passes=False),
        out_type=jax.ShapeDtypeStruct((batch,), jnp.float32),
        scratch_types=[
            pltpu.VMEM((b_per_w,), jnp.int32),
            pltpu.VMEM((b_per_w,), jnp.int32),
            pltpu.VMEM((b_per_w // _IDX_CHUNK, _IDX_CHUNK), jnp.int32),
            pltpu.VMEM((b_per_w // _IDX_CHUNK, _IDX_CHUNK), jnp.int32),
            pltpu.VMEM((2, _IDX_CHUNK, 128), jnp.float32),
            pltpu.VMEM((2, _IDX_CHUNK, 128), jnp.float32),
            pltpu.VMEM((b_per_w,), jnp.float32),
            pltpu.SemaphoreType.DMA,
            pltpu.SemaphoreType.DMA,
        ],
    )
    return run(user_ids.astype(jnp.int32), movie_ids.astype(jnp.int32),
               ut, mt)


# speedup vs baseline: 1.5926x; 1.5926x over previous
import functools

import jax
import jax.numpy as jnp
from jax import lax
from jax.experimental import pallas as pl
from jax.experimental.pallas import tpu as pltpu
from jax.experimental.pallas import tpu_sc as plsc

_EMBED = 32
_HALF = 256


def _dot_kernel(uid_hbm, mid_hbm, utab_hbm, mtab_hbm, out_hbm,
                uid_v, mid_v, du_v, dm_v, out_v, sem,
                *, b_per_w, num_cores):
    wid = lax.axis_index("s") * num_cores + lax.axis_index("c")
    base = wid * b_per_w

    pltpu.sync_copy(uid_hbm.at[pl.ds(base, b_per_w)], uid_v)
    pltpu.sync_copy(mid_hbm.at[pl.ds(base, b_per_w)], mid_v)

    lane = lax.iota(jnp.int32, 16)

    for p in range(b_per_w // _HALF):
        p0 = p * _HALF

        def fire(g, _):
            k0 = g * 16
            rv = uid_v[pl.ds(p0 + k0, 16)]
            rm = mid_v[pl.ds(p0 + k0, 16)]
            for k in range(16):
                pltpu.async_copy(utab_hbm.at[pl.ds(rv[k], 1), :],
                                 du_v.at[pl.ds(k0 + k, 1), :], sem)
                pltpu.async_copy(mtab_hbm.at[pl.ds(rm[k], 1), :],
                                 dm_v.at[pl.ds(k0 + k, 1), :], sem)
            return 0

        lax.fori_loop(0, _HALF // 16, fire, 0)

        pltpu.make_async_copy(utab_hbm.at[pl.ds(0, _HALF), :], du_v,
                              sem).wait()
        pltpu.make_async_copy(mtab_hbm.at[pl.ds(0, _HALF), :], dm_v,
                              sem).wait()

        def group(g, _):
            k0 = g * 16
            acc = jnp.zeros((16,), jnp.float32)
            for k in range(16):
                row = k0 + k
                v = (du_v[row, pl.ds(0, 16)] * dm_v[row, pl.ds(0, 16)]
                     + du_v[row, pl.ds(16, 16)] * dm_v[row, pl.ds(16, 16)])
                acc = jnp.where(lane == k, jnp.sum(v), acc)
            out_v[pl.ds(p0 + k0, 16)] = acc
            return 0

        lax.fori_loop(0, _HALF // 16, group, 0)

    pltpu.sync_copy(out_v, out_hbm.at[pl.ds(base, b_per_w)])


def kernel(user_ids, movie_ids, user_table, movie_table):
    batch = user_ids.shape[0]
    info = plsc.get_sparse_core_info()
    nw = info.num_cores * info.num_subcores
    b_per_w = batch // nw
    mesh = plsc.VectorSubcoreMesh(core_axis_name="c", subcore_axis_name="s")

    run = pl.kernel(
        functools.partial(_dot_kernel, b_per_w=b_per_w,
                          num_cores=info.num_cores),
        mesh=mesh,
        compiler_params=pltpu.CompilerParams(needs_layout_passes=False),
        out_type=jax.ShapeDtypeStruct((batch,), jnp.float32),
        scratch_types=[
            pltpu.VMEM((b_per_w,), jnp.int32),
            pltpu.VMEM((b_per_w,), jnp.int32),
            pltpu.VMEM((_HALF, _EMBED), jnp.float32),
            pltpu.VMEM((_HALF, _EMBED), jnp.float32),
            pltpu.VMEM((b_per_w,), jnp.float32),
            pltpu.SemaphoreType.DMA,
        ],
    )
    return run(user_ids.astype(jnp.int32), movie_ids.astype(jnp.int32),
               user_table, movie_table)
